# Initial kernel scaffold; baseline (speedup 1.0000x reference)
import jax
import jax.numpy as jnp
from jax.experimental import pallas as pl


def kernel(x, edge_index, W1, b1, W2, b2):
    raise NotImplementedError("write your pallas kernel here")


# speedup vs baseline: 13.8938x; 13.8938x over previous
import functools

import jax
import jax.numpy as jnp
from jax import lax
from jax.experimental import pallas as pl
from jax.experimental.pallas import tpu as pltpu
from jax.experimental.pallas import tpu_sc as plsc

NC = 2
NS = 16
NW = NC * NS
C = 128
HW = 8


def _sc_mesh():
  return plsc.VectorSubcoreMesh(core_axis_name="c", subcore_axis_name="s")


def _make_hist(PN, K):
  rpt = PN // NS

  @functools.partial(
      pl.kernel,
      out_type=jax.ShapeDtypeStruct((NC, PN, HW), jnp.float32),
      mesh=_sc_mesh(),
      scratch_types=[
          pltpu.VMEM((K, C), jnp.int32),
          pltpu.VMEM((C, HW), jnp.float32),
          pltpu.VMEM_SHARED((PN, HW), jnp.float32),
      ],
  )
  def hist(col_hbm, ones_hbm, zeros_hbm, out_hbm, col_v, ones_v, acc):
    cid = lax.axis_index("c")
    sid = lax.axis_index("s")
    wid = cid * NS + sid
    pltpu.sync_copy(zeros_hbm.at[pl.ds(sid * rpt, rpt)],
                    acc.at[pl.ds(sid * rpt, rpt)])
    pltpu.sync_copy(ones_hbm, ones_v)
    pltpu.sync_copy(col_hbm.at[wid], col_v)
    plsc.subcore_barrier()

    def body(j, carry):
      pltpu.sync_copy(ones_v, acc.at[col_v.at[j]], add=True)
      return carry

    lax.fori_loop(0, K, body, 0)
    plsc.subcore_barrier()
    pltpu.sync_copy(acc.at[pl.ds(sid * rpt, rpt)],
                    out_hbm.at[cid, pl.ds(sid * rpt, rpt)])

  return hist


def _make_agg(PN, K, D):
  rpt = PN // NS

  @functools.partial(
      pl.kernel,
      out_type=jax.ShapeDtypeStruct((NC, PN, D), jnp.float32),
      mesh=_sc_mesh(),
      scratch_types=[
          pltpu.VMEM((K, C), jnp.int32),
          pltpu.VMEM((K, C), jnp.int32),
          pltpu.VMEM((C, D), jnp.float32),
          pltpu.VMEM_SHARED((PN, D), jnp.float32),
          pltpu.SemaphoreType.DMA,
      ],
  )
  def agg(xw_hbm, row_hbm, col_hbm, zeros_hbm, out_hbm,
          row_v, col_v, buf, acc, sem):
    cid = lax.axis_index("c")
    sid = lax.axis_index("s")
    wid = cid * NS + sid
    pltpu.sync_copy(zeros_hbm.at[pl.ds(sid * rpt, rpt)],
                    acc.at[pl.ds(sid * rpt, rpt)])
    pltpu.sync_copy(row_hbm.at[wid], row_v)
    pltpu.sync_copy(col_hbm.at[wid], col_v)
    plsc.subcore_barrier()

    def body(j, carry):
      pltpu.async_copy(xw_hbm.at[row_v.at[j]], buf, sem).wait()
      pltpu.sync_copy(buf, acc.at[col_v.at[j]], add=True)
      return carry

    lax.fori_loop(0, K, body, 0)
    plsc.subcore_barrier()
    pltpu.sync_copy(acc.at[pl.ds(sid * rpt, rpt)],
                    out_hbm.at[cid, pl.ds(sid * rpt, rpt)])

  return agg


def _tc_matmul(x, w, bn):
  n, d_in = x.shape
  d_out = w.shape[1]

  def body(x_ref, w_ref, o_ref):
    o_ref[...] = jnp.dot(x_ref[...], w_ref[...],
                         preferred_element_type=jnp.float32)

  return pl.pallas_call(
      body,
      grid=(n // bn,),
      in_specs=[
          pl.BlockSpec((bn, d_in), lambda i: (i, 0)),
          pl.BlockSpec((d_in, d_out), lambda i: (0, 0)),
      ],
      out_specs=pl.BlockSpec((bn, d_out), lambda i: (i, 0)),
      out_shape=jax.ShapeDtypeStruct((n, d_out), jnp.float32),
  )(x, w)


def _tc_scale(hist, xw, bn):
  n, d = xw.shape
  pn = hist.shape[1]

  def body(h_ref, xw_ref, xwp_ref, dinv_ref):
    deg = h_ref[0, :, 0:1] + h_ref[1, :, 0:1] + 2.0
    dinv = jax.lax.rsqrt(deg)
    db = jnp.broadcast_to(dinv, xw_ref.shape)
    dinv_ref[...] = db
    xwp_ref[...] = db * xw_ref[...]

  return pl.pallas_call(
      body,
      grid=(n // bn,),
      in_specs=[
          pl.BlockSpec((NC, bn, HW), lambda i: (0, i, 0)),
          pl.BlockSpec((bn, d), lambda i: (i, 0)),
      ],
      out_specs=[
          pl.BlockSpec((bn, d), lambda i: (i, 0)),
          pl.BlockSpec((bn, d), lambda i: (i, 0)),
      ],
      out_shape=[
          jax.ShapeDtypeStruct((n, d), jnp.float32),
          jax.ShapeDtypeStruct((n, d), jnp.float32),
      ],
  )(hist, xw)


def _tc_mid(agg, xwp, dinv, b1, w2, bn):
  n, d = xwp.shape
  pn = agg.shape[1]

  def body(a_ref, xwp_ref, dinv_ref, b_ref, w_ref, o_ref):
    s = a_ref[0] + a_ref[1] + 2.0 * xwp_ref[...]
    h = jnp.maximum(dinv_ref[...] * s + b_ref[...], 0.0)
    o_ref[...] = dinv_ref[...] * jnp.dot(
        h, w_ref[...], preferred_element_type=jnp.float32)

  return pl.pallas_call(
      body,
      grid=(n // bn,),
      in_specs=[
          pl.BlockSpec((NC, bn, d), lambda i: (0, i, 0)),
          pl.BlockSpec((bn, d), lambda i: (i, 0)),
          pl.BlockSpec((bn, d), lambda i: (i, 0)),
          pl.BlockSpec((1, d), lambda i: (0, 0)),
          pl.BlockSpec((d, d), lambda i: (0, 0)),
      ],
      out_specs=pl.BlockSpec((bn, d), lambda i: (i, 0)),
      out_shape=jax.ShapeDtypeStruct((n, d), jnp.float32),
  )(agg, xwp, dinv, b1, w2)


def _tc_final(agg, xwp, dinv, b2, bn):
  n, d = xwp.shape

  def body(a_ref, xwp_ref, dinv_ref, b_ref, o_ref):
    s = a_ref[0] + a_ref[1] + 2.0 * xwp_ref[...]
    o_ref[...] = dinv_ref[...] * s + b_ref[...]

  return pl.pallas_call(
      body,
      grid=(n // bn,),
      in_specs=[
          pl.BlockSpec((NC, bn, d), lambda i: (0, i, 0)),
          pl.BlockSpec((bn, d), lambda i: (i, 0)),
          pl.BlockSpec((bn, d), lambda i: (i, 0)),
          pl.BlockSpec((1, d), lambda i: (0, 0)),
      ],
      out_specs=pl.BlockSpec((bn, d), lambda i: (i, 0)),
      out_shape=jax.ShapeDtypeStruct((n, d), jnp.float32),
  )(agg, xwp, dinv, b2)


def kernel(x, edge_index, W1, b1, W2, b2):
  n, d = x.shape
  e = edge_index.shape[1]

  k = -(-e // (NW * C))
  e_pad = NW * k * C
  pn = -(-(n + 1) // (NS * 8)) * (NS * 8)
  dump = n
  bn = 1000

  row = edge_index[0].astype(jnp.int32)
  col = edge_index[1].astype(jnp.int32)
  row3 = jnp.concatenate(
      [row, jnp.zeros((e_pad - e,), jnp.int32)]).reshape(NW, k, C)
  col3 = jnp.concatenate(
      [col, jnp.full((e_pad - e,), dump, jnp.int32)]).reshape(NW, k, C)

  ones_hw = jnp.ones((C, HW), jnp.float32)
  zeros_hw = jnp.zeros((pn, HW), jnp.float32)
  zeros_d = jnp.zeros((pn, d), jnp.float32)

  hist = _make_hist(pn, k)(col3, ones_hw, zeros_hw)

  xw1 = _tc_matmul(x, W1, bn)
  xw1p, dinv = _tc_scale(hist, xw1, bn)

  agg_fn = _make_agg(pn, k, d)
  agg1 = agg_fn(xw1p, row3, col3, zeros_d)
  xw2p = _tc_mid(agg1, xw1p, dinv, b1.reshape(1, d), W2, bn)
  agg2 = agg_fn(xw2p, row3, col3, zeros_d)
  out = _tc_final(agg2, xw2p, dinv, b2.reshape(1, d), bn)
  return out
